# Initial kernel scaffold; baseline (speedup 1.0000x reference)
import jax
import jax.numpy as jnp
from jax.experimental import pallas as pl


def kernel(x1, x2, x3, edge_index1, edge_index2, edge_index3, edge_attr1, edge_attr2, edge_attr3, gc1_W, gc1_b, gc4_W, gc4_b, cW1, cb1, cW2, cb2):
    raise NotImplementedError("write your pallas kernel here")


# speedup vs baseline: 50.0040x; 50.0040x over previous
import functools

import jax
import jax.numpy as jnp
from jax import lax
from jax.experimental import pallas as pl
from jax.experimental.pallas import tpu as pltpu
from jax.experimental.pallas import tpu_sc as plsc

G = 128
NB = 3 * G
N = 268
E = 4288
NF = 128
NH = 16
NPAD = 272
NW = 32
PER_W = NB // NW


def _rsqrt_newton(d):
    y = plsc.bitcast(d, jnp.int32)
    y = jnp.int32(0x5F3759DF) - lax.shift_right_logical(y, 1)
    r = plsc.bitcast(y, jnp.float32)
    hx = d * 0.5
    for _ in range(3):
        r = r * (1.5 - hx * r * r)
    return jnp.where(d > 0.0, r, 0.0)


def _sc_inner(rowv, colv, yv, accv, wnv, compute_wn, wv, degv, dinvv):
    zz = jnp.zeros((16,), jnp.float32)

    @pl.loop(0, NPAD)
    def _zero_acc(g):
        accv[pl.ds(g * 16, 16)] = zz

    if compute_wn:
        @pl.loop(0, NPAD // 16)
        def _zero_deg(g):
            degv[pl.ds(g * 16, 16)] = zz

        @pl.loop(0, E // 16)
        def _deg(g):
            r = rowv[pl.ds(g * 16, 16)]
            plsc.addupdate_scatter(degv, [r], wv[pl.ds(g * 16, 16)])

        @pl.loop(0, NPAD // 16)
        def _dinv(g):
            degs = degv[pl.ds(g * 16, 16)]
            dinvv[pl.ds(g * 16, 16)] = _rsqrt_newton(degs)

    @pl.loop(0, E // 16)
    def _edges(g):
        r = rowv[pl.ds(g * 16, 16)]
        cc = colv[pl.ds(g * 16, 16)]
        if compute_wn:
            ww = wv[pl.ds(g * 16, 16)]
            dr = plsc.load_gather(dinvv, [r])
            dc = plsc.load_gather(dinvv, [cc])
            wn = -(dr * ww * dc)
            wnv[pl.ds(g * 16, 16)] = wn
        else:
            wn = wnv[pl.ds(g * 16, 16)]
        rb = lax.shift_left(r, 4)
        cb = lax.shift_left(cc, 4)
        for d in range(16):
            v = plsc.load_gather(yv, [rb + d])
            plsc.addupdate_scatter(accv, [cb + d], v * wn)


def _sc1_body(row_h, col_h, w_h, y_h, lh_h, wn_h,
              rowv, colv, wv, yv, accv, degv, dinvv, wnv):
    wid = lax.axis_index("s") * 2 + lax.axis_index("c")

    @pl.loop(0, PER_W)
    def _per_instance(k):
        i = wid * PER_W + k
        pltpu.sync_copy(row_h.at[i], rowv)
        pltpu.sync_copy(col_h.at[i], colv)
        pltpu.sync_copy(w_h.at[i], wv)
        pltpu.sync_copy(y_h.at[i], yv)
        _sc_inner(rowv, colv, yv, accv, wnv, True, wv, degv, dinvv)
        pltpu.sync_copy(accv.at[pl.ds(0, N * NH)], lh_h.at[i])
        pltpu.sync_copy(wnv, wn_h.at[i])


def _sc2_body(row_h, col_h, wnin_h, y_h, lh_h,
              rowv, colv, yv, accv, wnv):
    wid = lax.axis_index("s") * 2 + lax.axis_index("c")

    @pl.loop(0, PER_W)
    def _per_instance(k):
        i = wid * PER_W + k
        pltpu.sync_copy(row_h.at[i], rowv)
        pltpu.sync_copy(col_h.at[i], colv)
        pltpu.sync_copy(wnin_h.at[i], wnv)
        pltpu.sync_copy(y_h.at[i], yv)
        _sc_inner(rowv, colv, yv, accv, wnv, False, None, None, None)
        pltpu.sync_copy(accv.at[pl.ds(0, N * NH)], lh_h.at[i])


def _sc_mesh():
    return plsc.VectorSubcoreMesh(core_axis_name="c", subcore_axis_name="s",
                                  num_cores=2, num_subcores=16)


def _sc_pass1(row, col, ea, y):
    return pl.kernel(
        _sc1_body,
        out_type=(
            jax.ShapeDtypeStruct((NB, N * NH), jnp.float32),
            jax.ShapeDtypeStruct((NB, E), jnp.float32),
        ),
        mesh=_sc_mesh(),
        scratch_types=[
            pltpu.VMEM((E,), jnp.int32),
            pltpu.VMEM((E,), jnp.int32),
            pltpu.VMEM((E,), jnp.float32),
            pltpu.VMEM((N * NH,), jnp.float32),
            pltpu.VMEM((NPAD * NH,), jnp.float32),
            pltpu.VMEM((NPAD,), jnp.float32),
            pltpu.VMEM((NPAD,), jnp.float32),
            pltpu.VMEM((E,), jnp.float32),
        ],
        compiler_params=pltpu.CompilerParams(needs_layout_passes=False, use_tc_tiling_on_sc=False),
        name="sc_cheb_s1",
    )(row, col, ea, y)


def _sc_pass2(row, col, wn, y):
    return pl.kernel(
        _sc2_body,
        out_type=jax.ShapeDtypeStruct((NB, N * NH), jnp.float32),
        mesh=_sc_mesh(),
        scratch_types=[
            pltpu.VMEM((E,), jnp.int32),
            pltpu.VMEM((E,), jnp.int32),
            pltpu.VMEM((N * NH,), jnp.float32),
            pltpu.VMEM((NPAD * NH,), jnp.float32),
            pltpu.VMEM((E,), jnp.float32),
        ],
        compiler_params=pltpu.CompilerParams(needs_layout_passes=False, use_tc_tiling_on_sc=False),
        name="sc_cheb_s2",
    )(row, col, wn, y)


def _proj_body(x_ref, w_ref, b_ref, o0_ref, o1_ref, *, relu_in):
    x = x_ref[...]
    if relu_in is not None:
        x = jnp.maximum(x + relu_in[...], 0.0)
    acc = jnp.dot(x, w_ref[...], preferred_element_type=jnp.float32)
    acc = acc + b_ref[0:1, :]
    o0_ref[...] = acc[:, :NH]
    o1_ref[...] = acc[:, NH:]


def _proj1_body(x_ref, w_ref, b_ref, o0_ref, o1_ref):
    _proj_body(x_ref, w_ref, b_ref, o0_ref, o1_ref, relu_in=None)


def _proj2_body(x_ref, l_ref, w_ref, b_ref, o0_ref, o1_ref):
    _proj_body(x_ref, w_ref, b_ref, o0_ref, o1_ref, relu_in=l_ref)


def _tc_proj1(x2d, wcat, bcat):
    m = x2d.shape[0]
    bm = 4288
    grid = m // bm
    return pl.pallas_call(
        _proj1_body,
        grid=(grid,),
        in_specs=[
            pl.BlockSpec((bm, NF), lambda i: (i, 0)),
            pl.BlockSpec((NF, 2 * NH), lambda i: (0, 0)),
            pl.BlockSpec((8, 2 * NH), lambda i: (0, 0)),
        ],
        out_specs=[
            pl.BlockSpec((bm, NH), lambda i: (i, 0)),
            pl.BlockSpec((bm, NH), lambda i: (i, 0)),
        ],
        out_shape=[
            jax.ShapeDtypeStruct((m, NH), jnp.float32),
            jax.ShapeDtypeStruct((m, NH), jnp.float32),
        ],
    )(x2d, wcat, bcat)


def _tc_proj2(xw0b, lh, wcat, bcat):
    m = xw0b.shape[0]
    bm = 6432
    grid = m // bm
    return pl.pallas_call(
        _proj2_body,
        grid=(grid,),
        in_specs=[
            pl.BlockSpec((bm, NH), lambda i: (i, 0)),
            pl.BlockSpec((bm, NH), lambda i: (i, 0)),
            pl.BlockSpec((NH, 2 * NH), lambda i: (0, 0)),
            pl.BlockSpec((8, 2 * NH), lambda i: (0, 0)),
        ],
        out_specs=[
            pl.BlockSpec((bm, NH), lambda i: (i, 0)),
            pl.BlockSpec((bm, NH), lambda i: (i, 0)),
        ],
        out_shape=[
            jax.ShapeDtypeStruct((m, NH), jnp.float32),
            jax.ShapeDtypeStruct((m, NH), jnp.float32),
        ],
    )(xw0b, lh, wcat, bcat)


def _cls_body(a_ref, l_ref, w1_ref, b1_ref, w2_ref, b2_ref, o_ref):
    f = a_ref[...] + l_ref[...]
    z = jnp.dot(f, w1_ref[...], preferred_element_type=jnp.float32)
    z = jnp.maximum(z + b1_ref[0:1, :], 0.0)
    o = jnp.dot(z, w2_ref[...], preferred_element_type=jnp.float32)
    o_ref[...] = o + b2_ref[0:1, :]


def _tc_classifier(h2a, lh2, w1, b1, w2, b2):
    bm = 128
    grid = NB // bm
    return pl.pallas_call(
        _cls_body,
        grid=(grid,),
        in_specs=[
            pl.BlockSpec((bm, N * NH), lambda i: (i, 0)),
            pl.BlockSpec((bm, N * NH), lambda i: (i, 0)),
            pl.BlockSpec((N * NH, 1024), lambda i: (0, 0)),
            pl.BlockSpec((8, 1024), lambda i: (0, 0)),
            pl.BlockSpec((1024, 512), lambda i: (0, 0)),
            pl.BlockSpec((8, 512), lambda i: (0, 0)),
        ],
        out_specs=pl.BlockSpec((bm, 512), lambda i: (i, 0)),
        out_shape=jax.ShapeDtypeStruct((NB, 512), jnp.float32),
    )(h2a, lh2, w1, b1, w2, b2)


def kernel(x1, x2, x3, edge_index1, edge_index2, edge_index3,
           edge_attr1, edge_attr2, edge_attr3,
           gc1_W, gc1_b, gc4_W, gc4_b, cW1, cb1, cW2, cb2):
    x = jnp.concatenate([x1, x2, x3], axis=0)
    ei = jnp.concatenate([edge_index1, edge_index2, edge_index3], axis=0)
    ea = jnp.concatenate([edge_attr1, edge_attr2, edge_attr3], axis=0)
    row = ei[:, 0, :].astype(jnp.int32)
    col = ei[:, 1, :].astype(jnp.int32)
    ea = ea.astype(jnp.float32)

    x2d = x.reshape(NB * N, NF)

    wcat1 = jnp.concatenate([gc1_W[0], gc1_W[1]], axis=1)
    bcat1 = jnp.broadcast_to(
        jnp.concatenate([gc1_b, jnp.zeros_like(gc1_b)])[None, :], (8, 2 * NH))
    wcat4 = jnp.concatenate([gc4_W[0], gc4_W[1]], axis=1)
    bcat4 = jnp.broadcast_to(
        jnp.concatenate([gc4_b, jnp.zeros_like(gc4_b)])[None, :], (8, 2 * NH))
    b1r = jnp.broadcast_to(cb1[None, :], (8, 1024))
    b2r = jnp.broadcast_to(cb2[None, :], (8, 512))

    xw0b, xw1 = _tc_proj1(x2d, wcat1, bcat1)

    lh, wn = _sc_pass1(row, col, ea, xw1.reshape(NB, N * NH))

    hw0b, hw1 = _tc_proj2(xw0b, lh.reshape(NB * N, NH), wcat4, bcat4)

    lh2 = _sc_pass2(row, col, wn, hw1.reshape(NB, N * NH))

    out = _tc_classifier(hw0b.reshape(NB, N * NH), lh2, cW1, b1r, cW2, b2r)
    return (out[:G], out[G:2 * G], out[2 * G:])


# speedup vs baseline: 72.1809x; 1.4435x over previous
import functools

import jax
import jax.numpy as jnp
from jax import lax
from jax.experimental import pallas as pl
from jax.experimental.pallas import tpu as pltpu
from jax.experimental.pallas import tpu_sc as plsc

G = 128
NB = 3 * G
N = 268
E = 4288
NF = 128
NH = 16
NPAD = 272
NW = 32
PER_W = NB // NW


def _rsqrt_newton(d):
    y = plsc.bitcast(d, jnp.int32)
    y = jnp.int32(0x5F3759DF) - lax.shift_right_logical(y, 1)
    r = plsc.bitcast(y, jnp.float32)
    hx = d * 0.5
    for _ in range(3):
        r = r * (1.5 - hx * r * r)
    return jnp.where(d > 0.0, r, 0.0)


def _sc_inner(rowv, colv, yv, accv, wnv, compute_wn, wv, degv, dinvv):
    zz = jnp.zeros((16,), jnp.float32)

    @pl.loop(0, NPAD)
    def _zero_acc(g):
        accv[pl.ds(g * 16, 16)] = zz

    if compute_wn:
        @pl.loop(0, NPAD // 16)
        def _zero_deg(g):
            degv[pl.ds(g * 16, 16)] = zz

        @pl.loop(0, E // 16)
        def _deg(g):
            r = rowv[pl.ds(g * 16, 16)]
            plsc.addupdate_scatter(degv, [r], wv[pl.ds(g * 16, 16)])

        @pl.loop(0, NPAD // 16)
        def _dinv(g):
            degs = degv[pl.ds(g * 16, 16)]
            dinvv[pl.ds(g * 16, 16)] = _rsqrt_newton(degs)

    @pl.loop(0, E // 16)
    def _edges(g):
        r = rowv[pl.ds(g * 16, 16)]
        cc = colv[pl.ds(g * 16, 16)]
        if compute_wn:
            ww = wv[pl.ds(g * 16, 16)]
            dr = plsc.load_gather(dinvv, [r])
            dc = plsc.load_gather(dinvv, [cc])
            wn = -(dr * ww * dc)
            wnv[pl.ds(g * 16, 16)] = wn
        else:
            wn = wnv[pl.ds(g * 16, 16)]
        rb = lax.shift_left(r, 4)
        cb = lax.shift_left(cc, 4)
        vals = [plsc.load_gather(yv, [rb + d]) * wn for d in range(16)]
        for d in range(16):
            plsc.addupdate_scatter(accv, [cb + d], vals[d])


def _sc1_body(row_h, col_h, w_h, y_h, lh_h, wn_h,
              rowv, colv, wv, yv, accv, degv, dinvv, wnv):
    wid = lax.axis_index("s") * 2 + lax.axis_index("c")

    @pl.loop(0, PER_W)
    def _per_instance(k):
        i = wid * PER_W + k
        pltpu.sync_copy(row_h.at[i], rowv)
        pltpu.sync_copy(col_h.at[i], colv)
        pltpu.sync_copy(w_h.at[i], wv)
        pltpu.sync_copy(y_h.at[i], yv)
        _sc_inner(rowv, colv, yv, accv, wnv, True, wv, degv, dinvv)
        pltpu.sync_copy(accv.at[pl.ds(0, N * NH)], lh_h.at[i])
        pltpu.sync_copy(wnv, wn_h.at[i])


def _sc2_body(row_h, col_h, wnin_h, y_h, lh_h,
              rowv, colv, yv, accv, wnv):
    wid = lax.axis_index("s") * 2 + lax.axis_index("c")

    @pl.loop(0, PER_W)
    def _per_instance(k):
        i = wid * PER_W + k
        pltpu.sync_copy(row_h.at[i], rowv)
        pltpu.sync_copy(col_h.at[i], colv)
        pltpu.sync_copy(wnin_h.at[i], wnv)
        pltpu.sync_copy(y_h.at[i], yv)
        _sc_inner(rowv, colv, yv, accv, wnv, False, None, None, None)
        pltpu.sync_copy(accv.at[pl.ds(0, N * NH)], lh_h.at[i])


def _sc_mesh():
    return plsc.VectorSubcoreMesh(core_axis_name="c", subcore_axis_name="s",
                                  num_cores=2, num_subcores=16)


def _sc_pass1(row, col, ea, y):
    return pl.kernel(
        _sc1_body,
        out_type=(
            jax.ShapeDtypeStruct((NB, N * NH), jnp.float32),
            jax.ShapeDtypeStruct((NB, E), jnp.float32),
        ),
        mesh=_sc_mesh(),
        scratch_types=[
            pltpu.VMEM((E,), jnp.int32),
            pltpu.VMEM((E,), jnp.int32),
            pltpu.VMEM((E,), jnp.float32),
            pltpu.VMEM((N * NH,), jnp.float32),
            pltpu.VMEM((NPAD * NH,), jnp.float32),
            pltpu.VMEM((NPAD,), jnp.float32),
            pltpu.VMEM((NPAD,), jnp.float32),
            pltpu.VMEM((E,), jnp.float32),
        ],
        compiler_params=pltpu.CompilerParams(needs_layout_passes=False, use_tc_tiling_on_sc=False),
        name="sc_cheb_s1",
    )(row, col, ea, y)


def _sc_pass2(row, col, wn, y):
    return pl.kernel(
        _sc2_body,
        out_type=jax.ShapeDtypeStruct((NB, N * NH), jnp.float32),
        mesh=_sc_mesh(),
        scratch_types=[
            pltpu.VMEM((E,), jnp.int32),
            pltpu.VMEM((E,), jnp.int32),
            pltpu.VMEM((N * NH,), jnp.float32),
            pltpu.VMEM((NPAD * NH,), jnp.float32),
            pltpu.VMEM((E,), jnp.float32),
        ],
        compiler_params=pltpu.CompilerParams(needs_layout_passes=False, use_tc_tiling_on_sc=False),
        name="sc_cheb_s2",
    )(row, col, wn, y)


def _proj_body(x_ref, w_ref, b_ref, o0_ref, o1_ref, *, relu_in):
    x = x_ref[...]
    if relu_in is not None:
        x = jnp.maximum(x + relu_in[...], 0.0)
    acc = jnp.dot(x, w_ref[...], preferred_element_type=jnp.float32)
    acc = acc + b_ref[0:1, :]
    o0_ref[...] = acc[:, :NH]
    o1_ref[...] = acc[:, NH:]


def _proj1_body(x_ref, w_ref, b_ref, o0_ref, o1_ref):
    _proj_body(x_ref, w_ref, b_ref, o0_ref, o1_ref, relu_in=None)


def _proj2_body(x_ref, l_ref, w_ref, b_ref, o0_ref, o1_ref):
    _proj_body(x_ref, w_ref, b_ref, o0_ref, o1_ref, relu_in=l_ref)


def _tc_proj1(x2d, wcat, bcat):
    m = x2d.shape[0]
    bm = 4288
    grid = m // bm
    return pl.pallas_call(
        _proj1_body,
        grid=(grid,),
        in_specs=[
            pl.BlockSpec((bm, NF), lambda i: (i, 0)),
            pl.BlockSpec((NF, 2 * NH), lambda i: (0, 0)),
            pl.BlockSpec((8, 2 * NH), lambda i: (0, 0)),
        ],
        out_specs=[
            pl.BlockSpec((bm, NH), lambda i: (i, 0)),
            pl.BlockSpec((bm, NH), lambda i: (i, 0)),
        ],
        out_shape=[
            jax.ShapeDtypeStruct((m, NH), jnp.float32),
            jax.ShapeDtypeStruct((m, NH), jnp.float32),
        ],
    )(x2d, wcat, bcat)


def _tc_proj2(xw0b, lh, wcat, bcat):
    m = xw0b.shape[0]
    bm = 6432
    grid = m // bm
    return pl.pallas_call(
        _proj2_body,
        grid=(grid,),
        in_specs=[
            pl.BlockSpec((bm, NH), lambda i: (i, 0)),
            pl.BlockSpec((bm, NH), lambda i: (i, 0)),
            pl.BlockSpec((NH, 2 * NH), lambda i: (0, 0)),
            pl.BlockSpec((8, 2 * NH), lambda i: (0, 0)),
        ],
        out_specs=[
            pl.BlockSpec((bm, NH), lambda i: (i, 0)),
            pl.BlockSpec((bm, NH), lambda i: (i, 0)),
        ],
        out_shape=[
            jax.ShapeDtypeStruct((m, NH), jnp.float32),
            jax.ShapeDtypeStruct((m, NH), jnp.float32),
        ],
    )(xw0b, lh, wcat, bcat)


def _cls_body(a_ref, l_ref, w1_ref, b1_ref, w2_ref, b2_ref, o_ref):
    f = a_ref[...] + l_ref[...]
    z = jnp.dot(f, w1_ref[...], preferred_element_type=jnp.float32)
    z = jnp.maximum(z + b1_ref[0:1, :], 0.0)
    o = jnp.dot(z, w2_ref[...], preferred_element_type=jnp.float32)
    o_ref[...] = o + b2_ref[0:1, :]


def _tc_classifier(h2a, lh2, w1, b1, w2, b2):
    bm = 128
    grid = NB // bm
    return pl.pallas_call(
        _cls_body,
        grid=(grid,),
        in_specs=[
            pl.BlockSpec((bm, N * NH), lambda i: (i, 0)),
            pl.BlockSpec((bm, N * NH), lambda i: (i, 0)),
            pl.BlockSpec((N * NH, 1024), lambda i: (0, 0)),
            pl.BlockSpec((8, 1024), lambda i: (0, 0)),
            pl.BlockSpec((1024, 512), lambda i: (0, 0)),
            pl.BlockSpec((8, 512), lambda i: (0, 0)),
        ],
        out_specs=pl.BlockSpec((bm, 512), lambda i: (i, 0)),
        out_shape=jax.ShapeDtypeStruct((NB, 512), jnp.float32),
    )(h2a, lh2, w1, b1, w2, b2)


def kernel(x1, x2, x3, edge_index1, edge_index2, edge_index3,
           edge_attr1, edge_attr2, edge_attr3,
           gc1_W, gc1_b, gc4_W, gc4_b, cW1, cb1, cW2, cb2):
    x = jnp.concatenate([x1, x2, x3], axis=0)
    ei = jnp.concatenate([edge_index1, edge_index2, edge_index3], axis=0)
    ea = jnp.concatenate([edge_attr1, edge_attr2, edge_attr3], axis=0)
    row = ei[:, 0, :].astype(jnp.int32)
    col = ei[:, 1, :].astype(jnp.int32)
    ea = ea.astype(jnp.float32)

    x2d = x.reshape(NB * N, NF)

    wcat1 = jnp.concatenate([gc1_W[0], gc1_W[1]], axis=1)
    bcat1 = jnp.broadcast_to(
        jnp.concatenate([gc1_b, jnp.zeros_like(gc1_b)])[None, :], (8, 2 * NH))
    wcat4 = jnp.concatenate([gc4_W[0], gc4_W[1]], axis=1)
    bcat4 = jnp.broadcast_to(
        jnp.concatenate([gc4_b, jnp.zeros_like(gc4_b)])[None, :], (8, 2 * NH))
    b1r = jnp.broadcast_to(cb1[None, :], (8, 1024))
    b2r = jnp.broadcast_to(cb2[None, :], (8, 512))

    xw0b, xw1 = _tc_proj1(x2d, wcat1, bcat1)

    lh, wn = _sc_pass1(row, col, ea, xw1.reshape(NB, N * NH))

    hw0b, hw1 = _tc_proj2(xw0b, lh.reshape(NB * N, NH), wcat4, bcat4)

    lh2 = _sc_pass2(row, col, wn, hw1.reshape(NB, N * NH))

    out = _tc_classifier(hw0b.reshape(NB, N * NH), lh2, cW1, b1r, cW2, b2r)
    return (out[:G], out[G:2 * G], out[2 * G:])


# speedup vs baseline: 117.5026x; 1.6279x over previous
import functools

import jax
import jax.numpy as jnp
from jax import lax
from jax.experimental import pallas as pl
from jax.experimental.pallas import tpu as pltpu
from jax.experimental.pallas import tpu_sc as plsc

G = 128
NB = 3 * G
N = 268
E = 4288
NF = 128
NH = 16
NPAD = 272
NW = 32
PER_W = NB // NW


def _rsqrt_newton(d):
    y = plsc.bitcast(d, jnp.int32)
    y = jnp.int32(0x5F3759DF) - lax.shift_right_logical(y, 1)
    r = plsc.bitcast(y, jnp.float32)
    hx = d * 0.5
    for _ in range(3):
        r = r * (1.5 - hx * r * r)
    return jnp.where(d > 0.0, r, 0.0)


SKEW = 289
TSIZE = SKEW * NH + 16


def _sc_inner(rowv, colv, ybuf, yT, accT, obuf, wnv, compute_wn, wv, degv, dinvv):
    zz = jnp.zeros((16,), jnp.float32)
    iota = lax.iota(jnp.int32, 16)
    tidx = iota * SKEW

    @pl.loop(0, TSIZE // 16)
    def _zero_acc(g):
        accT[pl.ds(g * 16, 16)] = zz

    @pl.loop(0, N)
    def _tin(n):
        v = ybuf[pl.ds(n * 16, 16)]
        plsc.store_scatter(yT, [tidx + n], v)

    if compute_wn:
        @pl.loop(0, NPAD // 16)
        def _zero_deg(g):
            degv[pl.ds(g * 16, 16)] = zz

        @pl.loop(0, E // 16)
        def _deg(g):
            r = rowv[pl.ds(g * 16, 16)]
            plsc.addupdate_scatter(degv, [r], wv[pl.ds(g * 16, 16)])

        @pl.loop(0, NPAD // 16)
        def _dinv(g):
            degs = degv[pl.ds(g * 16, 16)]
            dinvv[pl.ds(g * 16, 16)] = _rsqrt_newton(degs)

    @pl.loop(0, E // 16)
    def _edges(g):
        r = rowv[pl.ds(g * 16, 16)]
        cc = colv[pl.ds(g * 16, 16)]
        if compute_wn:
            ww = wv[pl.ds(g * 16, 16)]
            dr = plsc.load_gather(dinvv, [r])
            dc = plsc.load_gather(dinvv, [cc])
            wn = -(dr * ww * dc)
            wnv[pl.ds(g * 16, 16)] = wn
        else:
            wn = wnv[pl.ds(g * 16, 16)]
        vals = [plsc.load_gather(yT, [r + SKEW * d]) * wn for d in range(16)]
        for d in range(16):
            plsc.addupdate_scatter(accT, [cc + SKEW * d], vals[d])

    @pl.loop(0, N)
    def _tout(n):
        v = plsc.load_gather(accT, [tidx + n])
        obuf[pl.ds(n * 16, 16)] = v


def _sc1_body(row_h, col_h, w_h, y_h, lh_h, wn_h,
              rowv, colv, wv, ybuf, yT, accT, obuf, degv, dinvv, wnv):
    wid = lax.axis_index("s") * 2 + lax.axis_index("c")

    @pl.loop(0, PER_W)
    def _per_instance(k):
        i = wid * PER_W + k
        pltpu.sync_copy(row_h.at[i], rowv)
        pltpu.sync_copy(col_h.at[i], colv)
        pltpu.sync_copy(w_h.at[i], wv)
        pltpu.sync_copy(y_h.at[i], ybuf)
        _sc_inner(rowv, colv, ybuf, yT, accT, obuf, wnv, True, wv, degv, dinvv)
        pltpu.sync_copy(obuf, lh_h.at[i])
        pltpu.sync_copy(wnv, wn_h.at[i])


def _sc2_body(row_h, col_h, wnin_h, y_h, lh_h,
              rowv, colv, ybuf, yT, accT, obuf, wnv):
    wid = lax.axis_index("s") * 2 + lax.axis_index("c")

    @pl.loop(0, PER_W)
    def _per_instance(k):
        i = wid * PER_W + k
        pltpu.sync_copy(row_h.at[i], rowv)
        pltpu.sync_copy(col_h.at[i], colv)
        pltpu.sync_copy(wnin_h.at[i], wnv)
        pltpu.sync_copy(y_h.at[i], ybuf)
        _sc_inner(rowv, colv, ybuf, yT, accT, obuf, wnv, False, None, None, None)
        pltpu.sync_copy(obuf, lh_h.at[i])


def _sc_mesh():
    return plsc.VectorSubcoreMesh(core_axis_name="c", subcore_axis_name="s",
                                  num_cores=2, num_subcores=16)


def _sc_pass1(row, col, ea, y):
    return pl.kernel(
        _sc1_body,
        out_type=(
            jax.ShapeDtypeStruct((NB, N * NH), jnp.float32),
            jax.ShapeDtypeStruct((NB, E), jnp.float32),
        ),
        mesh=_sc_mesh(),
        scratch_types=[
            pltpu.VMEM((E,), jnp.int32),
            pltpu.VMEM((E,), jnp.int32),
            pltpu.VMEM((E,), jnp.float32),
            pltpu.VMEM((N * NH,), jnp.float32),
            pltpu.VMEM((TSIZE,), jnp.float32),
            pltpu.VMEM((TSIZE,), jnp.float32),
            pltpu.VMEM((N * NH,), jnp.float32),
            pltpu.VMEM((NPAD,), jnp.float32),
            pltpu.VMEM((NPAD,), jnp.float32),
            pltpu.VMEM((E,), jnp.float32),
        ],
        compiler_params=pltpu.CompilerParams(needs_layout_passes=False, use_tc_tiling_on_sc=False),
        name="sc_cheb_s1",
    )(row, col, ea, y)


def _sc_pass2(row, col, wn, y):
    return pl.kernel(
        _sc2_body,
        out_type=jax.ShapeDtypeStruct((NB, N * NH), jnp.float32),
        mesh=_sc_mesh(),
        scratch_types=[
            pltpu.VMEM((E,), jnp.int32),
            pltpu.VMEM((E,), jnp.int32),
            pltpu.VMEM((N * NH,), jnp.float32),
            pltpu.VMEM((TSIZE,), jnp.float32),
            pltpu.VMEM((TSIZE,), jnp.float32),
            pltpu.VMEM((N * NH,), jnp.float32),
            pltpu.VMEM((E,), jnp.float32),
        ],
        compiler_params=pltpu.CompilerParams(needs_layout_passes=False, use_tc_tiling_on_sc=False),
        name="sc_cheb_s2",
    )(row, col, wn, y)


def _proj_body(x_ref, w_ref, b_ref, o0_ref, o1_ref, *, relu_in):
    x = x_ref[...]
    if relu_in is not None:
        x = jnp.maximum(x + relu_in[...], 0.0)
    acc = jnp.dot(x, w_ref[...], preferred_element_type=jnp.float32)
    acc = acc + b_ref[0:1, :]
    o0_ref[...] = acc[:, :NH]
    o1_ref[...] = acc[:, NH:]


def _proj1_body(x_ref, w_ref, b_ref, o0_ref, o1_ref):
    _proj_body(x_ref, w_ref, b_ref, o0_ref, o1_ref, relu_in=None)


def _proj2_body(x_ref, l_ref, w_ref, b_ref, o0_ref, o1_ref):
    _proj_body(x_ref, w_ref, b_ref, o0_ref, o1_ref, relu_in=l_ref)


def _tc_proj1(x2d, wcat, bcat):
    m = x2d.shape[0]
    bm = 4288
    grid = m // bm
    return pl.pallas_call(
        _proj1_body,
        grid=(grid,),
        in_specs=[
            pl.BlockSpec((bm, NF), lambda i: (i, 0)),
            pl.BlockSpec((NF, 2 * NH), lambda i: (0, 0)),
            pl.BlockSpec((8, 2 * NH), lambda i: (0, 0)),
        ],
        out_specs=[
            pl.BlockSpec((bm, NH), lambda i: (i, 0)),
            pl.BlockSpec((bm, NH), lambda i: (i, 0)),
        ],
        out_shape=[
            jax.ShapeDtypeStruct((m, NH), jnp.float32),
            jax.ShapeDtypeStruct((m, NH), jnp.float32),
        ],
    )(x2d, wcat, bcat)


def _tc_proj2(xw0b, lh, wcat, bcat):
    m = xw0b.shape[0]
    bm = 6432
    grid = m // bm
    return pl.pallas_call(
        _proj2_body,
        grid=(grid,),
        in_specs=[
            pl.BlockSpec((bm, NH), lambda i: (i, 0)),
            pl.BlockSpec((bm, NH), lambda i: (i, 0)),
            pl.BlockSpec((NH, 2 * NH), lambda i: (0, 0)),
            pl.BlockSpec((8, 2 * NH), lambda i: (0, 0)),
        ],
        out_specs=[
            pl.BlockSpec((bm, NH), lambda i: (i, 0)),
            pl.BlockSpec((bm, NH), lambda i: (i, 0)),
        ],
        out_shape=[
            jax.ShapeDtypeStruct((m, NH), jnp.float32),
            jax.ShapeDtypeStruct((m, NH), jnp.float32),
        ],
    )(xw0b, lh, wcat, bcat)


def _cls_body(a_ref, l_ref, w1_ref, b1_ref, w2_ref, b2_ref, o_ref):
    f = a_ref[...] + l_ref[...]
    z = jnp.dot(f, w1_ref[...], preferred_element_type=jnp.float32)
    z = jnp.maximum(z + b1_ref[0:1, :], 0.0)
    o = jnp.dot(z, w2_ref[...], preferred_element_type=jnp.float32)
    o_ref[...] = o + b2_ref[0:1, :]


def _tc_classifier(h2a, lh2, w1, b1, w2, b2):
    bm = 128
    grid = NB // bm
    return pl.pallas_call(
        _cls_body,
        grid=(grid,),
        in_specs=[
            pl.BlockSpec((bm, N * NH), lambda i: (i, 0)),
            pl.BlockSpec((bm, N * NH), lambda i: (i, 0)),
            pl.BlockSpec((N * NH, 1024), lambda i: (0, 0)),
            pl.BlockSpec((8, 1024), lambda i: (0, 0)),
            pl.BlockSpec((1024, 512), lambda i: (0, 0)),
            pl.BlockSpec((8, 512), lambda i: (0, 0)),
        ],
        out_specs=pl.BlockSpec((bm, 512), lambda i: (i, 0)),
        out_shape=jax.ShapeDtypeStruct((NB, 512), jnp.float32),
    )(h2a, lh2, w1, b1, w2, b2)


def kernel(x1, x2, x3, edge_index1, edge_index2, edge_index3,
           edge_attr1, edge_attr2, edge_attr3,
           gc1_W, gc1_b, gc4_W, gc4_b, cW1, cb1, cW2, cb2):
    x = jnp.concatenate([x1, x2, x3], axis=0)
    ei = jnp.concatenate([edge_index1, edge_index2, edge_index3], axis=0)
    ea = jnp.concatenate([edge_attr1, edge_attr2, edge_attr3], axis=0)
    row = ei[:, 0, :].astype(jnp.int32)
    col = ei[:, 1, :].astype(jnp.int32)
    ea = ea.astype(jnp.float32)

    x2d = x.reshape(NB * N, NF)

    wcat1 = jnp.concatenate([gc1_W[0], gc1_W[1]], axis=1)
    bcat1 = jnp.broadcast_to(
        jnp.concatenate([gc1_b, jnp.zeros_like(gc1_b)])[None, :], (8, 2 * NH))
    wcat4 = jnp.concatenate([gc4_W[0], gc4_W[1]], axis=1)
    bcat4 = jnp.broadcast_to(
        jnp.concatenate([gc4_b, jnp.zeros_like(gc4_b)])[None, :], (8, 2 * NH))
    b1r = jnp.broadcast_to(cb1[None, :], (8, 1024))
    b2r = jnp.broadcast_to(cb2[None, :], (8, 512))

    xw0b, xw1 = _tc_proj1(x2d, wcat1, bcat1)

    lh, wn = _sc_pass1(row, col, ea, xw1.reshape(NB, N * NH))

    hw0b, hw1 = _tc_proj2(xw0b, lh.reshape(NB * N, NH), wcat4, bcat4)

    lh2 = _sc_pass2(row, col, wn, hw1.reshape(NB, N * NH))

    out = _tc_classifier(hw0b.reshape(NB, N * NH), lh2, cW1, b1r, cW2, b2r)
    return (out[:G], out[G:2 * G], out[2 * G:])


# speedup vs baseline: 122.3278x; 1.0411x over previous
import functools

import jax
import jax.numpy as jnp
from jax import lax
from jax.experimental import pallas as pl
from jax.experimental.pallas import tpu as pltpu
from jax.experimental.pallas import tpu_sc as plsc

G = 128
NB = 3 * G
N = 268
E = 4288
NF = 128
NH = 16
NPAD = 272
NW = 32
PER_W = NB // NW


def _rsqrt_newton(d):
    y = plsc.bitcast(d, jnp.int32)
    y = jnp.int32(0x5F3759DF) - lax.shift_right_logical(y, 1)
    r = plsc.bitcast(y, jnp.float32)
    hx = d * 0.5
    for _ in range(3):
        r = r * (1.5 - hx * r * r)
    return jnp.where(d > 0.0, r, 0.0)


SKEW = 289
TSIZE = SKEW * NH + 16


def _sc_inner(rowv, colv, ybuf, yT, accT, obuf, wnv, compute_wn, wv, degv, dinvv):
    zz = jnp.zeros((16,), jnp.float32)
    iota = lax.iota(jnp.int32, 16)
    tidx = iota * SKEW

    @pl.loop(0, TSIZE // 16)
    def _zero_acc(g):
        accT[pl.ds(g * 16, 16)] = zz

    @pl.loop(0, N)
    def _tin(n):
        v = ybuf[pl.ds(n * 16, 16)]
        plsc.store_scatter(yT, [tidx + n], v)

    if compute_wn:
        @pl.loop(0, NPAD // 16)
        def _zero_deg(g):
            degv[pl.ds(g * 16, 16)] = zz

        @pl.loop(0, E // 16)
        def _deg(g):
            r = rowv[pl.ds(g * 16, 16)]
            plsc.addupdate_scatter(degv, [r], wv[pl.ds(g * 16, 16)])

        @pl.loop(0, NPAD // 16)
        def _dinv(g):
            degs = degv[pl.ds(g * 16, 16)]
            dinvv[pl.ds(g * 16, 16)] = _rsqrt_newton(degs)

    @pl.loop(0, E // 16)
    def _edges(g):
        r = rowv[pl.ds(g * 16, 16)]
        cc = colv[pl.ds(g * 16, 16)]
        if compute_wn:
            ww = wv[pl.ds(g * 16, 16)]
            dr = plsc.load_gather(dinvv, [r])
            dc = plsc.load_gather(dinvv, [cc])
            wn = -(dr * ww * dc)
            wnv[pl.ds(g * 16, 16)] = wn
        else:
            wn = wnv[pl.ds(g * 16, 16)]
        vals = [plsc.load_gather(yT, [r + SKEW * d]) * wn for d in range(16)]
        for d in range(16):
            plsc.addupdate_scatter(accT, [cc + SKEW * d], vals[d])

    @pl.loop(0, N)
    def _tout(n):
        v = plsc.load_gather(accT, [tidx + n])
        obuf[pl.ds(n * 16, 16)] = v


PER_G = G // NW


def _sc1_body(e1, e2, e3, a1, a2, a3, y1, y2, y3,
              l1, l2, l3, wn1, wn2, wn3,
              rowv, colv, wv, ybuf, yT, accT, obuf, degv, dinvv, wnv):
    wid = lax.axis_index("s") * 2 + lax.axis_index("c")
    for eh, ah, yh, lhh, wnh in ((e1, a1, y1, l1, wn1),
                                 (e2, a2, y2, l2, wn2),
                                 (e3, a3, y3, l3, wn3)):
        @pl.loop(0, PER_G)
        def _per_instance(k):
            g = wid * PER_G + k
            pltpu.sync_copy(eh.at[g, 0], rowv)
            pltpu.sync_copy(eh.at[g, 1], colv)
            pltpu.sync_copy(ah.at[g], wv)
            pltpu.sync_copy(yh.at[g], ybuf)
            _sc_inner(rowv, colv, ybuf, yT, accT, obuf, wnv, True, wv, degv, dinvv)
            pltpu.sync_copy(obuf, lhh.at[g])
            pltpu.sync_copy(wnv, wnh.at[g])


def _sc2_body(e1, e2, e3, w1, w2, w3, y1, y2, y3,
              l1, l2, l3,
              rowv, colv, ybuf, yT, accT, obuf, wnv):
    wid = lax.axis_index("s") * 2 + lax.axis_index("c")
    for eh, wh, yh, lhh in ((e1, w1, y1, l1),
                            (e2, w2, y2, l2),
                            (e3, w3, y3, l3)):
        @pl.loop(0, PER_G)
        def _per_instance(k):
            g = wid * PER_G + k
            pltpu.sync_copy(eh.at[g, 0], rowv)
            pltpu.sync_copy(eh.at[g, 1], colv)
            pltpu.sync_copy(wh.at[g], wnv)
            pltpu.sync_copy(yh.at[g], ybuf)
            _sc_inner(rowv, colv, ybuf, yT, accT, obuf, wnv, False, None, None, None)
            pltpu.sync_copy(obuf, lhh.at[g])


def _sc_mesh():
    return plsc.VectorSubcoreMesh(core_axis_name="c", subcore_axis_name="s",
                                  num_cores=2, num_subcores=16)


def _sc_pass1(eis, eas, ys):
    lh_t = jax.ShapeDtypeStruct((G, N * NH), jnp.float32)
    wn_t = jax.ShapeDtypeStruct((G, E), jnp.float32)
    return pl.kernel(
        _sc1_body,
        out_type=(lh_t, lh_t, lh_t, wn_t, wn_t, wn_t),
        mesh=_sc_mesh(),
        scratch_types=[
            pltpu.VMEM((E,), jnp.int32),
            pltpu.VMEM((E,), jnp.int32),
            pltpu.VMEM((E,), jnp.float32),
            pltpu.VMEM((N * NH,), jnp.float32),
            pltpu.VMEM((TSIZE,), jnp.float32),
            pltpu.VMEM((TSIZE,), jnp.float32),
            pltpu.VMEM((N * NH,), jnp.float32),
            pltpu.VMEM((NPAD,), jnp.float32),
            pltpu.VMEM((NPAD,), jnp.float32),
            pltpu.VMEM((E,), jnp.float32),
        ],
        compiler_params=pltpu.CompilerParams(needs_layout_passes=False, use_tc_tiling_on_sc=False),
        name="sc_cheb_s1",
    )(*eis, *eas, *ys)


def _sc_pass2(eis, wns, ys):
    lh_t = jax.ShapeDtypeStruct((G, N * NH), jnp.float32)
    return pl.kernel(
        _sc2_body,
        out_type=(lh_t, lh_t, lh_t),
        mesh=_sc_mesh(),
        scratch_types=[
            pltpu.VMEM((E,), jnp.int32),
            pltpu.VMEM((E,), jnp.int32),
            pltpu.VMEM((N * NH,), jnp.float32),
            pltpu.VMEM((TSIZE,), jnp.float32),
            pltpu.VMEM((TSIZE,), jnp.float32),
            pltpu.VMEM((N * NH,), jnp.float32),
            pltpu.VMEM((E,), jnp.float32),
        ],
        compiler_params=pltpu.CompilerParams(needs_layout_passes=False, use_tc_tiling_on_sc=False),
        name="sc_cheb_s2",
    )(*eis, *wns, *ys)


def _proj_body(x_ref, w_ref, b_ref, o0_ref, o1_ref, *, relu_in):
    x = x_ref[...]
    if relu_in is not None:
        x = jnp.maximum(x + relu_in[...], 0.0)
    acc = jnp.dot(x, w_ref[...], preferred_element_type=jnp.float32)
    acc = acc + b_ref[0:1, :]
    o0_ref[...] = acc[:, :NH]
    o1_ref[...] = acc[:, NH:]


def _proj1_body(x_ref, w_ref, b_ref, o0_ref, o1_ref):
    _proj_body(x_ref, w_ref, b_ref, o0_ref, o1_ref, relu_in=None)


def _proj2_body(x_ref, l_ref, w_ref, b_ref, o0_ref, o1_ref):
    _proj_body(x_ref, w_ref, b_ref, o0_ref, o1_ref, relu_in=l_ref)


def _tc_proj1(x2d, wcat, bcat):
    m = x2d.shape[0]
    bm = 4288
    grid = m // bm
    return pl.pallas_call(
        _proj1_body,
        grid=(grid,),
        in_specs=[
            pl.BlockSpec((bm, NF), lambda i: (i, 0)),
            pl.BlockSpec((NF, 2 * NH), lambda i: (0, 0)),
            pl.BlockSpec((8, 2 * NH), lambda i: (0, 0)),
        ],
        out_specs=[
            pl.BlockSpec((bm, NH), lambda i: (i, 0)),
            pl.BlockSpec((bm, NH), lambda i: (i, 0)),
        ],
        out_shape=[
            jax.ShapeDtypeStruct((m, NH), jnp.float32),
            jax.ShapeDtypeStruct((m, NH), jnp.float32),
        ],
    )(x2d, wcat, bcat)


def _tc_proj2(xw0b, lh, wcat, bcat):
    m = xw0b.shape[0]
    bm = 4288
    grid = m // bm
    return pl.pallas_call(
        _proj2_body,
        grid=(grid,),
        in_specs=[
            pl.BlockSpec((bm, NH), lambda i: (i, 0)),
            pl.BlockSpec((bm, NH), lambda i: (i, 0)),
            pl.BlockSpec((NH, 2 * NH), lambda i: (0, 0)),
            pl.BlockSpec((8, 2 * NH), lambda i: (0, 0)),
        ],
        out_specs=[
            pl.BlockSpec((bm, NH), lambda i: (i, 0)),
            pl.BlockSpec((bm, NH), lambda i: (i, 0)),
        ],
        out_shape=[
            jax.ShapeDtypeStruct((m, NH), jnp.float32),
            jax.ShapeDtypeStruct((m, NH), jnp.float32),
        ],
    )(xw0b, lh, wcat, bcat)


def _cls_body(a_ref, l_ref, w1_ref, b1_ref, w2_ref, b2_ref, o_ref):
    f = a_ref[...] + l_ref[...]
    z = jnp.dot(f, w1_ref[...], preferred_element_type=jnp.float32)
    z = jnp.maximum(z + b1_ref[0:1, :], 0.0)
    o = jnp.dot(z, w2_ref[...], preferred_element_type=jnp.float32)
    o_ref[...] = o + b2_ref[0:1, :]


def _tc_classifier(h2a, lh2, w1, b1, w2, b2):
    bm = 128
    grid = G // bm
    return pl.pallas_call(
        _cls_body,
        grid=(grid,),
        in_specs=[
            pl.BlockSpec((bm, N * NH), lambda i: (i, 0)),
            pl.BlockSpec((bm, N * NH), lambda i: (i, 0)),
            pl.BlockSpec((N * NH, 1024), lambda i: (0, 0)),
            pl.BlockSpec((8, 1024), lambda i: (0, 0)),
            pl.BlockSpec((1024, 512), lambda i: (0, 0)),
            pl.BlockSpec((8, 512), lambda i: (0, 0)),
        ],
        out_specs=pl.BlockSpec((bm, 512), lambda i: (i, 0)),
        out_shape=jax.ShapeDtypeStruct((G, 512), jnp.float32),
    )(h2a, lh2, w1, b1, w2, b2)


def kernel(x1, x2, x3, edge_index1, edge_index2, edge_index3,
           edge_attr1, edge_attr2, edge_attr3,
           gc1_W, gc1_b, gc4_W, gc4_b, cW1, cb1, cW2, cb2):
    xs = (x1, x2, x3)
    eis = tuple(e.astype(jnp.int32) for e in
                (edge_index1, edge_index2, edge_index3))
    eas = (edge_attr1, edge_attr2, edge_attr3)

    wcat1 = jnp.concatenate([gc1_W[0], gc1_W[1]], axis=1)
    bcat1 = jnp.broadcast_to(
        jnp.concatenate([gc1_b, jnp.zeros_like(gc1_b)])[None, :], (8, 2 * NH))
    wcat4 = jnp.concatenate([gc4_W[0], gc4_W[1]], axis=1)
    bcat4 = jnp.broadcast_to(
        jnp.concatenate([gc4_b, jnp.zeros_like(gc4_b)])[None, :], (8, 2 * NH))
    b1r = jnp.broadcast_to(cb1[None, :], (8, 1024))
    b2r = jnp.broadcast_to(cb2[None, :], (8, 512))

    proj1 = [_tc_proj1(x.reshape(G * N, NF), wcat1, bcat1) for x in xs]
    xw0bs = [p[0] for p in proj1]
    xw1s = [p[1].reshape(G, N * NH) for p in proj1]

    l1, l2, l3, wn1, wn2, wn3 = _sc_pass1(eis, eas, xw1s)

    proj2 = [_tc_proj2(xw0bs[b], lh.reshape(G * N, NH), wcat4, bcat4)
             for b, lh in enumerate((l1, l2, l3))]
    hw0bs = [p[0].reshape(G, N * NH) for p in proj2]
    hw1s = [p[1].reshape(G, N * NH) for p in proj2]

    lh2s = _sc_pass2(eis, (wn1, wn2, wn3), hw1s)

    outs = [_tc_classifier(hw0bs[b], lh2s[b], cW1, b1r, cW2, b2r)
            for b in range(3)]
    return tuple(outs)
